# baseline (speedup 1.0000x reference)
import functools

import jax
import jax.numpy as jnp
from jax import lax
from jax.experimental import pallas as pl
from jax.experimental.pallas import tpu as pltpu
from jax.experimental.pallas import tpu_sc as plsc

_NC = 2
_NS = 16


@functools.partial(jax.jit, static_argnames=("n", "h", "nch", "chunk"))
def _sc_edge_agg(p, edge_index, *, n, h, nch, chunk):
    npad = -(-n // (8 * _NS)) * (8 * _NS)
    rows_per_tile = npad // _NS
    ew = nch * chunk
    mesh = plsc.VectorSubcoreMesh(core_axis_name="c", subcore_axis_name="s")
    ring = 10
    half = ring // 2
    assert (nch - half) % ring >= 0 and (nch - ring) % ring == 0

    @functools.partial(
        pl.kernel,
        out_type=jax.ShapeDtypeStruct((_NC, npad, h), jnp.float32),
        mesh=mesh,
        scratch_types=[
            pltpu.VMEM((ew,), jnp.int32),
            pltpu.VMEM((ew,), jnp.int32),
            pltpu.VMEM((ring, chunk, h), jnp.float32),
            pltpu.VMEM((rows_per_tile, h), jnp.float32),
            pltpu.VMEM_SHARED((npad, h), jnp.float32),
            pltpu.SemaphoreType.DMA((ring,)),
            pltpu.SemaphoreType.DMA((ring,)),
        ],
        compiler_params=pltpu.CompilerParams(use_tc_tiling_on_sc=False),
    )
    def body(p_hbm, edge_hbm, out_hbm, idxs_v, idxd_v, rows_v, slab_v,
             acc_sh, semg, sems):
        c = lax.axis_index("c")
        s = lax.axis_index("s")
        w = c * _NS + s
        base = pl.multiple_of(w * ew, 8)

        dsrc = pltpu.async_copy(edge_hbm.at[0].at[pl.ds(base, ew)], idxs_v,
                                semg.at[0])
        ddst = pltpu.async_copy(edge_hbm.at[1].at[pl.ds(base, ew)], idxd_v,
                                semg.at[1])

        zeros16 = jnp.zeros((16,), jnp.float32)

        def zero_rows(r, carry):
            for rr in range(4):
                for lo in range(0, h, 16):
                    slab_v[r * 4 + rr, pl.ds(lo, 16)] = zeros16
            return carry

        lax.fori_loop(0, rows_per_tile // 4, zero_rows, 0)
        dsrc.wait()
        ddst.wait()
        pltpu.sync_copy(slab_v, acc_sh.at[pl.ds(s * rows_per_tile,
                                                rows_per_tile)])
        plsc.subcore_barrier()

        def gather_src(j):
            off = pl.multiple_of(j * chunk, 8)
            return p_hbm.at[idxs_v.at[pl.ds(off, chunk)]]

        def scatter_dst(j):
            off = pl.multiple_of(j * chunk, 8)
            return acc_sh.at[idxd_v.at[pl.ds(off, chunk)]]

        def wait_gather(b):
            pltpu.make_async_copy(gather_src(0), rows_v.at[b],
                                  semg.at[b]).wait()

        def wait_scatter(b):
            pltpu.make_async_copy(rows_v.at[b], scatter_dst(0),
                                  sems.at[b]).wait()

        for b in range(half):
            pltpu.async_copy(gather_src(b), rows_v.at[b], semg.at[b])
        for jj in range(half):
            b = jj % ring
            b2 = (b + half) % ring
            wait_gather(b)
            pltpu.async_copy(rows_v.at[b], scatter_dst(jj), sems.at[b],
                             add=True)
            pltpu.async_copy(gather_src(jj + half), rows_v.at[b2],
                             semg.at[b2])

        def group(g, carry):
            for t in range(ring):
                j = half + g * ring + t
                b = (half + t) % ring
                b2 = (b + half) % ring
                wait_gather(b)
                pltpu.async_copy(rows_v.at[b], scatter_dst(j), sems.at[b],
                                 add=True)
                wait_scatter(b2)
                pltpu.async_copy(gather_src(j + half), rows_v.at[b2],
                                 semg.at[b2])
            return carry

        lax.fori_loop(0, (nch - ring) // ring, group, 0)

        for jj in range(nch - half, nch):
            b = jj % ring
            wait_gather(b)
            pltpu.async_copy(rows_v.at[b], scatter_dst(jj), sems.at[b],
                             add=True)
        for b in range(ring):
            wait_scatter(b)
        plsc.subcore_barrier()

        pltpu.sync_copy(acc_sh.at[pl.ds(s * rows_per_tile, rows_per_tile)],
                        slab_v)
        pltpu.sync_copy(
            slab_v,
            out_hbm.at[c].at[pl.ds(s * rows_per_tile, rows_per_tile)])

    return body(p, edge_index)


def _proj_body(x_ref, w_ref, o_ref):
    o_ref[...] = jnp.dot(x_ref[...], w_ref[...],
                         preferred_element_type=jnp.float32)


def _post(p_ref, a_ref, b1_ref, w2_ref, b2_ref, g_ref, be_ref):
    z = p_ref[...] + a_ref[0] + a_ref[1] + b1_ref[...]
    z = jnp.maximum(z, 0.0)
    z = jnp.dot(z, w2_ref[...], preferred_element_type=jnp.float32)
    z = jnp.maximum(z + b2_ref[...], 0.0)
    return z * g_ref[...] + be_ref[...]


def _mid_body(p_ref, a_ref, b1_ref, w2_ref, b2_ref, g_ref, be_ref, w1n_ref,
              o_ref):
    hcur = _post(p_ref, a_ref, b1_ref, w2_ref, b2_ref, g_ref, be_ref)
    o_ref[...] = jnp.dot(hcur, w1n_ref[...],
                         preferred_element_type=jnp.float32)


def _final_body(p_ref, a_ref, b1_ref, w2_ref, b2_ref, g_ref, be_ref,
                batch_ref, fc1w_ref, fc1b_ref, fc2w_ref, fc2b_ref,
                o_ref, acc_ref, *, n_graphs):
    i = pl.program_id(0)
    hcur = _post(p_ref, a_ref, b1_ref, w2_ref, b2_ref, g_ref, be_ref)
    blk = hcur.shape[0]
    onehot = (batch_ref[0, 0, :][:, None]
              == lax.broadcasted_iota(jnp.int32, (blk, n_graphs), 1)
              ).astype(jnp.float32)
    part = lax.dot_general(onehot, hcur, (((0,), (0,)), ((), ())),
                           preferred_element_type=jnp.float32)

    @pl.when(i == 0)
    def _():
        acc_ref[...] = part

    @pl.when(i > 0)
    def _():
        acc_ref[...] += part

    @pl.when(i == pl.num_programs(0) - 1)
    def _():
        u = jnp.dot(acc_ref[...], fc1w_ref[...],
                    preferred_element_type=jnp.float32) + fc1b_ref[...]
        u = jnp.maximum(u, 0.0)
        u = jnp.dot(u, fc2w_ref[...],
                    preferred_element_type=jnp.float32) + fc2b_ref[...]
        m = jnp.mean(u, axis=0, keepdims=True)
        mx = jnp.max(m)
        e = jnp.exp(m - mx)
        o_ref[...] = m - mx - jnp.log(jnp.sum(e))


def _full(shape):
    return pl.BlockSpec(shape, lambda i: (0,) * len(shape))


def kernel(x, edge_index, batch, W1_0, b1_0, W2_0, b2_0, g_0, be_0,
           W1_1, b1_1, W2_1, b2_1, g_1, be_1, W1_2, b1_2, W2_2, b2_2,
           g_2, be_2, fc1_w, fc1_b, fc2_w, fc2_b):
    n, d = x.shape
    h = W1_0.shape[1]
    n_graphs = 128
    c = fc2_w.shape[1]
    e = edge_index.shape[1]

    blk = 5000
    nb = n // blk

    nw = _NC * _NS
    ew = e // nw
    chunk = max(ck for ck in range(8, 129, 8)
                if ew % ck == 0 and (ew // ck) % 10 == 0)
    nch = ew // chunk

    b1s = [b1_0.reshape(1, h), b1_1.reshape(1, h), b1_2.reshape(1, h)]
    b2s = [b2_0.reshape(1, h), b2_1.reshape(1, h), b2_2.reshape(1, h)]
    gs = [g_0.reshape(1, h), g_1.reshape(1, h), g_2.reshape(1, h)]
    bes = [be_0.reshape(1, h), be_1.reshape(1, h), be_2.reshape(1, h)]
    w2s = [W2_0, W2_1, W2_2]
    batch3 = batch.reshape(nb, 1, blk)

    p = pl.pallas_call(
        _proj_body,
        grid=(nb,),
        in_specs=[pl.BlockSpec((blk, d), lambda i: (i, 0)),
                  _full((d, h))],
        out_specs=pl.BlockSpec((blk, h), lambda i: (i, 0)),
        out_shape=jax.ShapeDtypeStruct((n, h), jnp.float32),
    )(x, W1_0)

    vec_spec = [pl.BlockSpec((blk, h), lambda i: (i, 0)),
                pl.BlockSpec((_NC, blk, h), lambda i: (0, i, 0))]
    small = [_full((1, h)), _full((h, h)), _full((1, h)), _full((1, h)),
             _full((1, h))]

    for layer in range(2):
        agg = _sc_edge_agg(p, edge_index, n=n, h=h, nch=nch, chunk=chunk)
        p = pl.pallas_call(
            _mid_body,
            grid=(nb,),
            in_specs=vec_spec + small + [_full((h, h))],
            out_specs=pl.BlockSpec((blk, h), lambda i: (i, 0)),
            out_shape=jax.ShapeDtypeStruct((n, h), jnp.float32),
        )(p, agg, b1s[layer], w2s[layer], b2s[layer], gs[layer], bes[layer],
          [W1_1, W1_2][layer])

    agg = _sc_edge_agg(p, edge_index, n=n, h=h, nch=nch, chunk=chunk)
    out = pl.pallas_call(
        functools.partial(_final_body, n_graphs=n_graphs),
        grid=(nb,),
        in_specs=vec_spec + small
        + [pl.BlockSpec((1, 1, blk), lambda i: (i, 0, 0)),
           _full((h, h)), _full((1, h)), _full((h, c)), _full((1, c))],
        out_specs=_full((1, c)),
        out_shape=jax.ShapeDtypeStruct((1, c), jnp.float32),
        scratch_shapes=[pltpu.VMEM((n_graphs, h), jnp.float32)],
    )(p, agg, b1s[2], w2s[2], b2s[2], gs[2], bes[2], batch3,
      fc1_w, fc1_b.reshape(1, h), fc2_w, fc2_b.reshape(1, c))

    return out.reshape(c)


# speedup vs baseline: 1.1552x; 1.1552x over previous
import functools

import jax
import jax.numpy as jnp
from jax import lax
from jax.experimental import pallas as pl
from jax.experimental.pallas import tpu as pltpu
from jax.experimental.pallas import tpu_sc as plsc

_NC = 2
_NS = 16


@functools.partial(jax.jit, static_argnames=("n", "h", "nch", "chunk"))
def _sc_edge_agg(p, edge_index, *, n, h, nch, chunk):
    npad = -(-n // (8 * _NS)) * (8 * _NS)
    rows_per_tile = npad // _NS
    ew = nch * chunk
    prows = n // _NS
    mesh = plsc.VectorSubcoreMesh(core_axis_name="c", subcore_axis_name="s")
    ring = 10
    half = ring // 2
    steady_groups = (nch - ring) // ring
    tail_start = half + steady_groups * ring

    @functools.partial(
        pl.kernel,
        out_type=jax.ShapeDtypeStruct((_NC, npad, h), jnp.float32),
        mesh=mesh,
        scratch_types=[
            pltpu.VMEM((ew,), jnp.int32),
            pltpu.VMEM((ew,), jnp.int32),
            pltpu.VMEM((ring, chunk, h), jnp.float32),
            pltpu.VMEM((rows_per_tile, h), jnp.float32),
            pltpu.VMEM_SHARED((npad, h), jnp.float32),
            pltpu.VMEM_SHARED((n, h), jnp.float32),
            pltpu.SemaphoreType.DMA((ring,)),
            pltpu.SemaphoreType.DMA((ring,)),
            pltpu.SemaphoreType.DMA,
        ],
        compiler_params=pltpu.CompilerParams(use_tc_tiling_on_sc=False),
    )
    def body(p_hbm, edge_hbm, out_hbm, idxs_v, idxd_v, rows_v, slab_v,
             acc_sh, p_sh, semg, sems, semp):
        c = lax.axis_index("c")
        s = lax.axis_index("s")
        w = c * _NS + s
        base = pl.multiple_of(w * ew, 8)

        dsrc = pltpu.async_copy(edge_hbm.at[0].at[pl.ds(base, ew)], idxs_v,
                                semg.at[0])
        ddst = pltpu.async_copy(edge_hbm.at[1].at[pl.ds(base, ew)], idxd_v,
                                semg.at[1])
        dp = pltpu.async_copy(p_hbm.at[pl.ds(s * prows, prows)],
                              p_sh.at[pl.ds(s * prows, prows)], semp)

        zeros16 = jnp.zeros((16,), jnp.float32)

        def zero_rows(r, carry):
            for rr in range(4):
                for lo in range(0, h, 16):
                    slab_v[r * 4 + rr, pl.ds(lo, 16)] = zeros16
            return carry

        lax.fori_loop(0, rows_per_tile // 4, zero_rows, 0)
        dsrc.wait()
        ddst.wait()
        dp.wait()
        pltpu.sync_copy(slab_v, acc_sh.at[pl.ds(s * rows_per_tile,
                                                rows_per_tile)])
        plsc.subcore_barrier()

        def gather_src(j):
            off = pl.multiple_of(j * chunk, 8)
            return p_sh.at[idxs_v.at[pl.ds(off, chunk)]]

        def scatter_dst(j):
            off = pl.multiple_of(j * chunk, 8)
            return acc_sh.at[idxd_v.at[pl.ds(off, chunk)]]

        def wait_gather(b):
            pltpu.make_async_copy(gather_src(0), rows_v.at[b],
                                  semg.at[b]).wait()

        def wait_scatter(b):
            pltpu.make_async_copy(rows_v.at[b], scatter_dst(0),
                                  sems.at[b]).wait()

        for b in range(half):
            pltpu.async_copy(gather_src(b), rows_v.at[b], semg.at[b])
        for jj in range(half):
            b = jj % ring
            b2 = (b + half) % ring
            wait_gather(b)
            pltpu.async_copy(rows_v.at[b], scatter_dst(jj), sems.at[b],
                             add=True)
            pltpu.async_copy(gather_src(jj + half), rows_v.at[b2],
                             semg.at[b2])

        def group(g, carry):
            for t in range(ring):
                j = half + g * ring + t
                b = (half + t) % ring
                b2 = (b + half) % ring
                wait_gather(b)
                pltpu.async_copy(rows_v.at[b], scatter_dst(j), sems.at[b],
                                 add=True)
                wait_scatter(b2)
                pltpu.async_copy(gather_src(j + half), rows_v.at[b2],
                                 semg.at[b2])
            return carry

        lax.fori_loop(0, steady_groups, group, 0)

        for jj in range(tail_start, nch - half):
            b = jj % ring
            b2 = (b + half) % ring
            wait_gather(b)
            pltpu.async_copy(rows_v.at[b], scatter_dst(jj), sems.at[b],
                             add=True)
            wait_scatter(b2)
            pltpu.async_copy(gather_src(jj + half), rows_v.at[b2],
                             semg.at[b2])

        for jj in range(nch - half, nch):
            b = jj % ring
            wait_gather(b)
            pltpu.async_copy(rows_v.at[b], scatter_dst(jj), sems.at[b],
                             add=True)
        for b in range(ring):
            wait_scatter(b)
        plsc.subcore_barrier()

        pltpu.sync_copy(acc_sh.at[pl.ds(s * rows_per_tile, rows_per_tile)],
                        slab_v)
        pltpu.sync_copy(
            slab_v,
            out_hbm.at[c].at[pl.ds(s * rows_per_tile, rows_per_tile)])

    return body(p, edge_index)


def _proj_body(x_ref, w_ref, o_ref):
    o_ref[...] = jnp.dot(x_ref[...], w_ref[...],
                         preferred_element_type=jnp.float32)


def _post(p_ref, a_ref, b1_ref, w2_ref, b2_ref, g_ref, be_ref):
    z = p_ref[...] + a_ref[0] + a_ref[1] + b1_ref[...]
    z = jnp.maximum(z, 0.0)
    z = jnp.dot(z, w2_ref[...], preferred_element_type=jnp.float32)
    z = jnp.maximum(z + b2_ref[...], 0.0)
    return z * g_ref[...] + be_ref[...]


def _mid_body(p_ref, a_ref, b1_ref, w2_ref, b2_ref, g_ref, be_ref, w1n_ref,
              o_ref):
    hcur = _post(p_ref, a_ref, b1_ref, w2_ref, b2_ref, g_ref, be_ref)
    o_ref[...] = jnp.dot(hcur, w1n_ref[...],
                         preferred_element_type=jnp.float32)


def _final_body(p_ref, a_ref, b1_ref, w2_ref, b2_ref, g_ref, be_ref,
                batch_ref, fc1w_ref, fc1b_ref, fc2w_ref, fc2b_ref,
                o_ref, acc_ref, *, n_graphs):
    i = pl.program_id(0)
    hcur = _post(p_ref, a_ref, b1_ref, w2_ref, b2_ref, g_ref, be_ref)
    blk = hcur.shape[0]
    onehot = (batch_ref[0, 0, :][:, None]
              == lax.broadcasted_iota(jnp.int32, (blk, n_graphs), 1)
              ).astype(jnp.float32)
    part = lax.dot_general(onehot, hcur, (((0,), (0,)), ((), ())),
                           preferred_element_type=jnp.float32)

    @pl.when(i == 0)
    def _():
        acc_ref[...] = part

    @pl.when(i > 0)
    def _():
        acc_ref[...] += part

    @pl.when(i == pl.num_programs(0) - 1)
    def _():
        u = jnp.dot(acc_ref[...], fc1w_ref[...],
                    preferred_element_type=jnp.float32) + fc1b_ref[...]
        u = jnp.maximum(u, 0.0)
        u = jnp.dot(u, fc2w_ref[...],
                    preferred_element_type=jnp.float32) + fc2b_ref[...]
        m = jnp.mean(u, axis=0, keepdims=True)
        mx = jnp.max(m)
        e = jnp.exp(m - mx)
        o_ref[...] = m - mx - jnp.log(jnp.sum(e))


def _full(shape):
    return pl.BlockSpec(shape, lambda i: (0,) * len(shape))


def kernel(x, edge_index, batch, W1_0, b1_0, W2_0, b2_0, g_0, be_0,
           W1_1, b1_1, W2_1, b2_1, g_1, be_1, W1_2, b1_2, W2_2, b2_2,
           g_2, be_2, fc1_w, fc1_b, fc2_w, fc2_b):
    n, d = x.shape
    h = W1_0.shape[1]
    n_graphs = 128
    c = fc2_w.shape[1]
    e = edge_index.shape[1]

    blk = 5000
    nb = n // blk

    nw = _NC * _NS
    ew = e // nw
    chunk = max(ck for ck in range(8, 129, 8)
                if ew % ck == 0 and (ew // ck) % 5 == 0)
    nch = ew // chunk

    b1s = [b1_0.reshape(1, h), b1_1.reshape(1, h), b1_2.reshape(1, h)]
    b2s = [b2_0.reshape(1, h), b2_1.reshape(1, h), b2_2.reshape(1, h)]
    gs = [g_0.reshape(1, h), g_1.reshape(1, h), g_2.reshape(1, h)]
    bes = [be_0.reshape(1, h), be_1.reshape(1, h), be_2.reshape(1, h)]
    w2s = [W2_0, W2_1, W2_2]
    batch3 = batch.reshape(nb, 1, blk)

    p = pl.pallas_call(
        _proj_body,
        grid=(nb,),
        in_specs=[pl.BlockSpec((blk, d), lambda i: (i, 0)),
                  _full((d, h))],
        out_specs=pl.BlockSpec((blk, h), lambda i: (i, 0)),
        out_shape=jax.ShapeDtypeStruct((n, h), jnp.float32),
    )(x, W1_0)

    vec_spec = [pl.BlockSpec((blk, h), lambda i: (i, 0)),
                pl.BlockSpec((_NC, blk, h), lambda i: (0, i, 0))]
    small = [_full((1, h)), _full((h, h)), _full((1, h)), _full((1, h)),
             _full((1, h))]

    for layer in range(2):
        agg = _sc_edge_agg(p, edge_index, n=n, h=h, nch=nch, chunk=chunk)
        p = pl.pallas_call(
            _mid_body,
            grid=(nb,),
            in_specs=vec_spec + small + [_full((h, h))],
            out_specs=pl.BlockSpec((blk, h), lambda i: (i, 0)),
            out_shape=jax.ShapeDtypeStruct((n, h), jnp.float32),
        )(p, agg, b1s[layer], w2s[layer], b2s[layer], gs[layer], bes[layer],
          [W1_1, W1_2][layer])

    agg = _sc_edge_agg(p, edge_index, n=n, h=h, nch=nch, chunk=chunk)
    out = pl.pallas_call(
        functools.partial(_final_body, n_graphs=n_graphs),
        grid=(nb,),
        in_specs=vec_spec + small
        + [pl.BlockSpec((1, 1, blk), lambda i: (i, 0, 0)),
           _full((h, h)), _full((1, h)), _full((h, c)), _full((1, c))],
        out_specs=_full((1, c)),
        out_shape=jax.ShapeDtypeStruct((1, c), jnp.float32),
        scratch_shapes=[pltpu.VMEM((n_graphs, h), jnp.float32)],
    )(p, agg, b1s[2], w2s[2], b2s[2], gs[2], bes[2], batch3,
      fc1_w, fc1_b.reshape(1, h), fc2_w, fc2_b.reshape(1, c))

    return out.reshape(c)


# speedup vs baseline: 1.5645x; 1.3543x over previous
import functools

import jax
import jax.numpy as jnp
from jax import lax
from jax.experimental import pallas as pl
from jax.experimental.pallas import tpu as pltpu
from jax.experimental.pallas import tpu_sc as plsc

_NC = 2
_NS = 16


@functools.partial(jax.jit, static_argnames=("nch", "chunk"))
def _sc_edge_agg(p, edge_index, *, nch, chunk):
    npad, h = p.shape
    rows_per_tile = npad // _NS
    ew = nch * chunk
    mesh = plsc.VectorSubcoreMesh(core_axis_name="c", subcore_axis_name="s")
    ring = 10
    half = ring // 2
    steady_groups = (nch - ring) // ring
    tail_start = half + steady_groups * ring

    @functools.partial(
        pl.kernel,
        out_type=jax.ShapeDtypeStruct((_NC, npad, h), jnp.float32),
        mesh=mesh,
        scratch_types=[
            pltpu.VMEM((ew,), jnp.int32),
            pltpu.VMEM((ew,), jnp.int32),
            pltpu.VMEM((ring, chunk, h), jnp.float32),
            pltpu.VMEM((rows_per_tile, h), jnp.float32),
            pltpu.VMEM_SHARED((npad, h), jnp.float32),
            pltpu.VMEM_SHARED((npad, h), jnp.float32),
            pltpu.SemaphoreType.DMA((ring,)),
            pltpu.SemaphoreType.DMA((ring,)),
            pltpu.SemaphoreType.DMA,
        ],
        compiler_params=pltpu.CompilerParams(use_tc_tiling_on_sc=False),
    )
    def body(p_hbm, edge_hbm, out_hbm, idxs_v, idxd_v, rows_v, slab_v,
             acc_sh, p_sh, semg, sems, semp):
        c = lax.axis_index("c")
        s = lax.axis_index("s")
        w = c * _NS + s
        base = pl.multiple_of(w * ew, 8)

        dsrc = pltpu.async_copy(edge_hbm.at[0].at[pl.ds(base, ew)], idxs_v,
                                semg.at[0])
        ddst = pltpu.async_copy(edge_hbm.at[1].at[pl.ds(base, ew)], idxd_v,
                                semg.at[1])
        dp = pltpu.async_copy(p_hbm.at[pl.ds(s * rows_per_tile,
                                             rows_per_tile)],
                              p_sh.at[pl.ds(s * rows_per_tile,
                                            rows_per_tile)], semp)

        zeros16 = jnp.zeros((16,), jnp.float32)

        def zero_rows(r, carry):
            for rr in range(4):
                for lo in range(0, h, 16):
                    slab_v[r * 4 + rr, pl.ds(lo, 16)] = zeros16
            return carry

        lax.fori_loop(0, rows_per_tile // 4, zero_rows, 0)
        dsrc.wait()
        ddst.wait()
        dp.wait()
        pltpu.sync_copy(slab_v, acc_sh.at[pl.ds(s * rows_per_tile,
                                                rows_per_tile)])
        plsc.subcore_barrier()

        def gather_src(j):
            off = pl.multiple_of(j * chunk, 8)
            return p_sh.at[idxs_v.at[pl.ds(off, chunk)]]

        def scatter_dst(j):
            off = pl.multiple_of(j * chunk, 8)
            return acc_sh.at[idxd_v.at[pl.ds(off, chunk)]]

        def wait_gather(b):
            pltpu.make_async_copy(gather_src(0), rows_v.at[b],
                                  semg.at[b]).wait()

        def wait_scatter(b):
            pltpu.make_async_copy(rows_v.at[b], scatter_dst(0),
                                  sems.at[b]).wait()

        for b in range(half):
            pltpu.async_copy(gather_src(b), rows_v.at[b], semg.at[b])
        for jj in range(half):
            b = jj % ring
            b2 = (b + half) % ring
            wait_gather(b)
            pltpu.async_copy(rows_v.at[b], scatter_dst(jj), sems.at[b],
                             add=True)
            pltpu.async_copy(gather_src(jj + half), rows_v.at[b2],
                             semg.at[b2])

        def group(g, carry):
            for t in range(ring):
                j = half + g * ring + t
                b = (half + t) % ring
                b2 = (b + half) % ring
                wait_gather(b)
                pltpu.async_copy(rows_v.at[b], scatter_dst(j), sems.at[b],
                                 add=True)
                wait_scatter(b2)
                pltpu.async_copy(gather_src(j + half), rows_v.at[b2],
                                 semg.at[b2])
            return carry

        lax.fori_loop(0, steady_groups, group, 0)

        for jj in range(tail_start, nch - half):
            b = jj % ring
            b2 = (b + half) % ring
            wait_gather(b)
            pltpu.async_copy(rows_v.at[b], scatter_dst(jj), sems.at[b],
                             add=True)
            wait_scatter(b2)
            pltpu.async_copy(gather_src(jj + half), rows_v.at[b2],
                             semg.at[b2])

        for jj in range(nch - half, nch):
            b = jj % ring
            wait_gather(b)
            pltpu.async_copy(rows_v.at[b], scatter_dst(jj), sems.at[b],
                             add=True)
        for b in range(ring):
            wait_scatter(b)
        plsc.subcore_barrier()

        pltpu.sync_copy(acc_sh.at[pl.ds(s * rows_per_tile, rows_per_tile)],
                        slab_v)
        pltpu.sync_copy(
            slab_v,
            out_hbm.at[c].at[pl.ds(s * rows_per_tile, rows_per_tile)])

    return body(p, edge_index)


def _proj_body(x_ref, w_ref, o_ref, *, pack, pad_rows):
    xa = x_ref[...]
    nrows = xa.shape[0] // pack
    x3 = xa.reshape(nrows, pack, xa.shape[1])
    zs = [jnp.dot(x3[:, q, :], w_ref[...], preferred_element_type=jnp.float32)
          for q in range(pack)]
    z = jnp.concatenate(zs, axis=1)
    z = jnp.concatenate(
        [z, jnp.zeros((pad_rows, z.shape[1]), jnp.float32)], axis=0)
    o_ref[...] = z


def _post(p_ref, a_ref, b1_ref, w2_ref, b2_ref, g_ref, be_ref):
    z = p_ref[...] + a_ref[0] + a_ref[1] + b1_ref[...]
    z = jnp.maximum(z, 0.0)
    z = jnp.dot(z, w2_ref[...], preferred_element_type=jnp.float32)
    z = jnp.maximum(z + b2_ref[...], 0.0)
    return z * g_ref[...] + be_ref[...]


def _mid_body(p_ref, a_ref, b1_ref, w2_ref, b2_ref, g_ref, be_ref, w1n_ref,
              o_ref):
    hcur = _post(p_ref, a_ref, b1_ref, w2_ref, b2_ref, g_ref, be_ref)
    o_ref[...] = jnp.dot(hcur, w1n_ref[...],
                         preferred_element_type=jnp.float32)


def _final_body(p_ref, a_ref, b1_ref, w2_ref, b2_ref, g_ref, be_ref,
                batch_ref, fc1w_ref, fc1b_ref, fc2w_ref, fc2b_ref,
                o_ref, acc_ref, *, n_graphs, pack, hdim):
    i = pl.program_id(0)
    hcur = _post(p_ref, a_ref, b1_ref, w2_ref, b2_ref, g_ref, be_ref)
    blk4 = hcur.shape[0]
    bat = batch_ref[0]
    part = jnp.zeros((n_graphs, hdim), jnp.float32)
    for q in range(pack):
        onehot = (bat[:, q][:, None]
                  == lax.broadcasted_iota(jnp.int32, (blk4, n_graphs), 1)
                  ).astype(jnp.float32)
        hq = hcur[:, q * hdim:(q + 1) * hdim]
        part = part + lax.dot_general(onehot, hq, (((0,), (0,)), ((), ())),
                                      preferred_element_type=jnp.float32)

    @pl.when(i == 0)
    def _():
        acc_ref[...] = part

    @pl.when(i > 0)
    def _():
        acc_ref[...] += part

    @pl.when(i == pl.num_programs(0) - 1)
    def _():
        u = jnp.dot(acc_ref[...], fc1w_ref[...],
                    preferred_element_type=jnp.float32) + fc1b_ref[...]
        u = jnp.maximum(u, 0.0)
        u = jnp.dot(u, fc2w_ref[...],
                    preferred_element_type=jnp.float32) + fc2b_ref[...]
        m = jnp.mean(u, axis=0, keepdims=True)
        mx = jnp.max(m)
        e = jnp.exp(m - mx)
        o_ref[...] = m - mx - jnp.log(jnp.sum(e))


def _full(shape):
    return pl.BlockSpec(shape, lambda i: (0,) * len(shape))


def kernel(x, edge_index, batch, W1_0, b1_0, W2_0, b2_0, g_0, be_0,
           W1_1, b1_1, W2_1, b2_1, g_1, be_1, W1_2, b1_2, W2_2, b2_2,
           g_2, be_2, fc1_w, fc1_b, fc2_w, fc2_b):
    n, d = x.shape
    h = W1_0.shape[1]
    n_graphs = 128
    c = fc2_w.shape[1]
    e = edge_index.shape[1]
    pack = 128 // h
    npad = -(-n // (8 * _NS * pack)) * (8 * _NS * pack)
    n4 = npad // pack
    nb = 2
    blk4 = n4 // nb

    nw = _NC * _NS
    ew = e // nw
    chunk = max(ck for ck in range(8, 129, 8)
                if ew % ck == 0 and (ew // ck) % 5 == 0)
    nch = ew // chunk

    eye = jnp.eye(pack, dtype=jnp.float32)
    w2bd = [jnp.kron(eye, W2_0), jnp.kron(eye, W2_1), jnp.kron(eye, W2_2)]
    w1bd = [None, jnp.kron(eye, W1_1), jnp.kron(eye, W1_2)]
    b1t = [jnp.tile(v, pack).reshape(1, h * pack) for v in (b1_0, b1_1, b1_2)]
    b2t = [jnp.tile(v, pack).reshape(1, h * pack) for v in (b2_0, b2_1, b2_2)]
    gt = [jnp.tile(v, pack).reshape(1, h * pack) for v in (g_0, g_1, g_2)]
    bet = [jnp.tile(v, pack).reshape(1, h * pack) for v in (be_0, be_1, be_2)]
    batchp = jnp.pad(batch, (0, npad - n),
                     constant_values=n_graphs).reshape(nb, blk4, pack)

    p_pk = pl.pallas_call(
        functools.partial(_proj_body, pack=pack, pad_rows=n4 - n // pack),
        grid=(1,),
        in_specs=[_full((n, d)), _full((d, h))],
        out_specs=_full((n4, 128)),
        out_shape=jax.ShapeDtypeStruct((n4, 128), jnp.float32),
    )(x, W1_0)

    vec_spec = [pl.BlockSpec((blk4, 128), lambda i: (i, 0)),
                pl.BlockSpec((_NC, blk4, 128), lambda i: (0, i, 0))]
    small = [_full((1, 128)), _full((128, 128)), _full((1, 128)),
             _full((1, 128)), _full((1, 128))]

    for layer in range(2):
        agg = _sc_edge_agg(p_pk.reshape(npad, h), edge_index,
                           nch=nch, chunk=chunk)
        agg_pk = agg.reshape(_NC, n4, 128)
        p_pk = pl.pallas_call(
            _mid_body,
            grid=(nb,),
            in_specs=vec_spec + small + [_full((128, 128))],
            out_specs=pl.BlockSpec((blk4, 128), lambda i: (i, 0)),
            out_shape=jax.ShapeDtypeStruct((n4, 128), jnp.float32),
        )(p_pk, agg_pk, b1t[layer], w2bd[layer], b2t[layer], gt[layer],
          bet[layer], w1bd[layer + 1])

    agg = _sc_edge_agg(p_pk.reshape(npad, h), edge_index,
                       nch=nch, chunk=chunk)
    agg_pk = agg.reshape(_NC, n4, 128)
    out = pl.pallas_call(
        functools.partial(_final_body, n_graphs=n_graphs, pack=pack, hdim=h),
        grid=(nb,),
        in_specs=vec_spec + small
        + [pl.BlockSpec((1, blk4, pack), lambda i: (i, 0, 0)),
           _full((h, h)), _full((1, h)), _full((h, c)), _full((1, c))],
        out_specs=_full((1, c)),
        out_shape=jax.ShapeDtypeStruct((1, c), jnp.float32),
        scratch_shapes=[pltpu.VMEM((n_graphs, h), jnp.float32)],
    )(p_pk, agg_pk, b1t[2], w2bd[2], b2t[2], gt[2], bet[2], batchp,
      fc1_w, fc1_b.reshape(1, h), fc2_w, fc2_b.reshape(1, c))

    return out.reshape(c)
